# baseline (device time: 368036 ns/iter reference)
import os

import jax
import jax.numpy as jnp
from jax import lax
from jax.experimental import pallas as pl
from jax.experimental.pallas import tpu as pltpu

os.makedirs("/tmp/jax_scband_cache", exist_ok=True)
jax.config.update("jax_compilation_cache_dir", "/tmp/jax_scband_cache")
jax.config.update("jax_persistent_cache_min_compile_time_secs", 0.0)

N_DEV = 32
N_HOPS = N_DEV - 1
N_SLOTS = 3
N_SUB = 4

_PLANE_J = {(0, 0): 0, (1, 0): 1, (1, 1): 2, (0, 1): 3,
            (0, 2): 4, (1, 2): 5, (1, 3): 6, (0, 3): 7}

_YZ_PATH = [(y if z % 2 == 0 else 3 - y, z) for z in range(4) for y in range(4)]
_RING_COORDS = [(0, y, z) for (y, z) in _YZ_PATH] + \
               [(1, y, z) for (y, z) in reversed(_YZ_PATH)]
MESH_OF_RING = [8 * z + _PLANE_J[(x, y)] for (x, y, z) in _RING_COORDS]
RING_OF_MESH = [0] * N_DEV
for _k, _m in enumerate(MESH_OF_RING):
    RING_OF_MESH[_m] = _k
assert sorted(MESH_OF_RING) == list(range(N_DEV))


def kernel(x, w_mat):
    m_global, k_per = x.shape
    _, n = w_mat.shape
    m_per = m_global // N_DEV
    nh = n // 2
    m_sub = m_per // N_SUB

    xb = x.astype(jnp.bfloat16)
    wb = w_mat.astype(jnp.bfloat16)

    mesh_of_ring = jnp.asarray(MESH_OF_RING, jnp.int32)
    my_mesh = lax.axis_index("i")
    my_k = jnp.asarray(RING_OF_MESH, jnp.int32)[my_mesh]
    left_mesh = mesh_of_ring[(my_k + N_DEV - 1) % N_DEV]
    right_mesh = mesh_of_ring[(my_k + 1) % N_DEV]
    scalars = jnp.stack([my_k, left_mesh, right_mesh]).astype(jnp.int32)

    def body(scalar_ref, ring_tab_ref, x_ref, w_ref, out_ref,
             comm_r, comm_l, send_r, recv_r, send_l, recv_l,
             credit_r, credit_l):
        my_k = scalar_ref[0]
        left = scalar_ref[1]
        right = scalar_ref[2]

        barrier_sem = pltpu.get_barrier_semaphore()
        for nbr in (left, right):
            pl.semaphore_signal(
                barrier_sem, inc=1,
                device_id=(nbr,), device_id_type=pl.DeviceIdType.MESH,
            )
        pl.semaphore_wait(barrier_sem, 2)

        def partial(c, lo):
            row = ring_tab_ref[c]
            blk = x_ref[pl.ds(row * m_per, m_per), :]
            return jnp.dot(blk, w_ref[:, lo:lo + nh],
                           preferred_element_type=jnp.float32)

        def send_chunk_r(h):
            return (my_k + 2 * N_DEV - 1 - h) % N_DEV

        def send_chunk_l(h):
            return (my_k + 1 + h) % N_DEV

        def recv_chunk_r(h):
            return (my_k + 2 * N_DEV - 2 - h) % N_DEV

        def recv_chunk_l(h):
            return (my_k + 2 + h) % N_DEV

        lanes = [
            dict(comm=comm_r, send=send_r, recv=recv_r, credit=credit_r,
                 tgt=right, upstream=left, lo=0,
                 send_chunk=send_chunk_r, recv_chunk=recv_chunk_r),
            dict(comm=comm_l, send=send_l, recv=recv_l, credit=credit_l,
                 tgt=left, upstream=right, lo=nh,
                 send_chunk=send_chunk_l, recv_chunk=recv_chunk_l),
        ]

        def mk(ln, h, sub):
            s = h % N_SLOTS
            r = (h + 1) % N_SLOTS
            rows = pl.ds(sub * m_sub, m_sub)
            return pltpu.make_async_remote_copy(
                src_ref=ln["comm"].at[s, rows, :],
                dst_ref=ln["comm"].at[r, rows, :],
                send_sem=ln["send"].at[s, sub],
                recv_sem=ln["recv"].at[r, sub],
                device_id=(ln["tgt"],),
                device_id_type=pl.DeviceIdType.MESH,
            )

        descs = [[[None] * N_SUB for _ in range(N_HOPS)] for _ in lanes]

        for i, ln in enumerate(lanes):
            ln["comm"][0, :, :] = partial(ln["send_chunk"](0),
                                          ln["lo"]).astype(jnp.bfloat16)
            for sub in range(N_SUB):
                descs[i][0][sub] = mk(ln, 0, sub)
                descs[i][0][sub].start()
        p = [partial(ln["recv_chunk"](0), ln["lo"]) for ln in lanes]

        for h in range(1, N_HOPS):
            s = h % N_SLOTS
            for i, ln in enumerate(lanes):
                if h >= 2:
                    pl.semaphore_wait(ln["credit"], 1)
            for sub in range(N_SUB):
                rows = pl.ds(sub * m_sub, m_sub)
                rows_v = slice(sub * m_sub, (sub + 1) * m_sub)
                for i, ln in enumerate(lanes):
                    descs[i][h - 1][sub].wait_recv()
                    acc = (p[i][rows_v, :]
                           + ln["comm"][s, rows, :].astype(jnp.float32))
                    ln["comm"][s, rows, :] = acc.astype(jnp.bfloat16)
                    descs[i][h][sub] = mk(ln, h, sub)
                    descs[i][h][sub].start()
            for i, ln in enumerate(lanes):
                for sub in range(N_SUB):
                    descs[i][h - 1][sub].wait_send()
                if h <= N_HOPS - 1 and h <= 29:
                    pl.semaphore_signal(
                        ln["credit"], inc=1,
                        device_id=(ln["upstream"],),
                        device_id_type=pl.DeviceIdType.MESH,
                    )
            p = [partial(ln["recv_chunk"](h), ln["lo"]) for ln in lanes]

        fin = (N_HOPS - 1 + 1) % N_SLOTS
        for sub in range(N_SUB):
            rows = pl.ds(sub * m_sub, m_sub)
            rows_v = slice(sub * m_sub, (sub + 1) * m_sub)
            for i, ln in enumerate(lanes):
                descs[i][N_HOPS - 1][sub].wait_recv()
                out_ref[rows, ln["lo"]:ln["lo"] + nh] = (
                    p[i][rows_v, :]
                    + ln["comm"][fin, rows, :].astype(jnp.float32))
        for i, ln in enumerate(lanes):
            for sub in range(N_SUB):
                descs[i][N_HOPS - 1][sub].wait_send()

    return pl.pallas_call(
        body,
        out_shape=jax.ShapeDtypeStruct((m_per, n), jnp.float32),
        in_specs=[
            pl.BlockSpec(memory_space=pltpu.SMEM),
            pl.BlockSpec(memory_space=pltpu.SMEM),
            pl.BlockSpec(memory_space=pltpu.VMEM),
            pl.BlockSpec(memory_space=pltpu.VMEM),
        ],
        out_specs=pl.BlockSpec(memory_space=pltpu.VMEM),
        scratch_shapes=[
            pltpu.VMEM((N_SLOTS, m_per, nh), jnp.bfloat16),
            pltpu.VMEM((N_SLOTS, m_per, nh), jnp.bfloat16),
            pltpu.SemaphoreType.DMA((N_SLOTS, N_SUB)),
            pltpu.SemaphoreType.DMA((N_SLOTS, N_SUB)),
            pltpu.SemaphoreType.DMA((N_SLOTS, N_SUB)),
            pltpu.SemaphoreType.DMA((N_SLOTS, N_SUB)),
            pltpu.SemaphoreType.REGULAR,
            pltpu.SemaphoreType.REGULAR,
        ],
        compiler_params=pltpu.CompilerParams(collective_id=0),
    )(scalars, mesh_of_ring, xb, wb)


# device time: 367820 ns/iter; 1.0006x vs baseline; 1.0006x over previous
import os

import jax
import jax.numpy as jnp
from jax import lax
from jax.experimental import pallas as pl
from jax.experimental.pallas import tpu as pltpu

os.makedirs("/tmp/jax_scband_cache", exist_ok=True)
jax.config.update("jax_compilation_cache_dir", "/tmp/jax_scband_cache")
jax.config.update("jax_persistent_cache_min_compile_time_secs", 0.0)

N_DEV = 32
N_HOPS = N_DEV - 1
N_SLOTS = 3
N_SUB = 2

_PLANE_J = {(0, 0): 0, (1, 0): 1, (1, 1): 2, (0, 1): 3,
            (0, 2): 4, (1, 2): 5, (1, 3): 6, (0, 3): 7}

_YZ_PATH = [(y if z % 2 == 0 else 3 - y, z) for z in range(4) for y in range(4)]
_RING_COORDS = [(0, y, z) for (y, z) in _YZ_PATH] + \
               [(1, y, z) for (y, z) in reversed(_YZ_PATH)]
MESH_OF_RING = [8 * z + _PLANE_J[(x, y)] for (x, y, z) in _RING_COORDS]
RING_OF_MESH = [0] * N_DEV
for _k, _m in enumerate(MESH_OF_RING):
    RING_OF_MESH[_m] = _k
assert sorted(MESH_OF_RING) == list(range(N_DEV))


def kernel(x, w_mat):
    m_global, k_per = x.shape
    _, n = w_mat.shape
    m_per = m_global // N_DEV
    nh = n // 2
    m_sub = m_per // N_SUB

    xb = x.astype(jnp.bfloat16)
    wb = w_mat.astype(jnp.bfloat16)

    mesh_of_ring = jnp.asarray(MESH_OF_RING, jnp.int32)
    my_mesh = lax.axis_index("i")
    my_k = jnp.asarray(RING_OF_MESH, jnp.int32)[my_mesh]
    left_mesh = mesh_of_ring[(my_k + N_DEV - 1) % N_DEV]
    right_mesh = mesh_of_ring[(my_k + 1) % N_DEV]
    scalars = jnp.stack([my_k, left_mesh, right_mesh]).astype(jnp.int32)

    def body(scalar_ref, ring_tab_ref, x_ref, w_ref, out_ref,
             comm_r, comm_l, send_r, recv_r, send_l, recv_l,
             credit_r, credit_l):
        my_k = scalar_ref[0]
        left = scalar_ref[1]
        right = scalar_ref[2]

        barrier_sem = pltpu.get_barrier_semaphore()
        for nbr in (left, right):
            pl.semaphore_signal(
                barrier_sem, inc=1,
                device_id=(nbr,), device_id_type=pl.DeviceIdType.MESH,
            )
        pl.semaphore_wait(barrier_sem, 2)

        def partial(c, lo):
            row = ring_tab_ref[c]
            blk = x_ref[pl.ds(row * m_per, m_per), :]
            return jnp.dot(blk, w_ref[:, lo:lo + nh],
                           preferred_element_type=jnp.float32)

        def send_chunk_r(h):
            return (my_k + 2 * N_DEV - 1 - h) % N_DEV

        def send_chunk_l(h):
            return (my_k + 1 + h) % N_DEV

        def recv_chunk_r(h):
            return (my_k + 2 * N_DEV - 2 - h) % N_DEV

        def recv_chunk_l(h):
            return (my_k + 2 + h) % N_DEV

        lanes = [
            dict(comm=comm_r, send=send_r, recv=recv_r, credit=credit_r,
                 tgt=right, upstream=left, lo=0,
                 send_chunk=send_chunk_r, recv_chunk=recv_chunk_r),
            dict(comm=comm_l, send=send_l, recv=recv_l, credit=credit_l,
                 tgt=left, upstream=right, lo=nh,
                 send_chunk=send_chunk_l, recv_chunk=recv_chunk_l),
        ]

        def mk(ln, h, sub):
            s = h % N_SLOTS
            r = (h + 1) % N_SLOTS
            rows = pl.ds(sub * m_sub, m_sub)
            return pltpu.make_async_remote_copy(
                src_ref=ln["comm"].at[s, rows, :],
                dst_ref=ln["comm"].at[r, rows, :],
                send_sem=ln["send"].at[s, sub],
                recv_sem=ln["recv"].at[r, sub],
                device_id=(ln["tgt"],),
                device_id_type=pl.DeviceIdType.MESH,
            )

        descs = [[[None] * N_SUB for _ in range(N_HOPS)] for _ in lanes]

        for i, ln in enumerate(lanes):
            ln["comm"][0, :, :] = partial(ln["send_chunk"](0),
                                          ln["lo"]).astype(jnp.bfloat16)
            for sub in range(N_SUB):
                descs[i][0][sub] = mk(ln, 0, sub)
                descs[i][0][sub].start()
        p = [partial(ln["recv_chunk"](0), ln["lo"]) for ln in lanes]

        for h in range(1, N_HOPS):
            s = h % N_SLOTS
            for i, ln in enumerate(lanes):
                if h >= 2:
                    pl.semaphore_wait(ln["credit"], 1)
            for sub in range(N_SUB):
                rows = pl.ds(sub * m_sub, m_sub)
                rows_v = slice(sub * m_sub, (sub + 1) * m_sub)
                for i, ln in enumerate(lanes):
                    descs[i][h - 1][sub].wait_recv()
                    acc = (p[i][rows_v, :]
                           + ln["comm"][s, rows, :].astype(jnp.float32))
                    ln["comm"][s, rows, :] = acc.astype(jnp.bfloat16)
                    descs[i][h][sub] = mk(ln, h, sub)
                    descs[i][h][sub].start()
            for i, ln in enumerate(lanes):
                for sub in range(N_SUB):
                    descs[i][h - 1][sub].wait_send()
                if h <= N_HOPS - 1 and h <= 29:
                    pl.semaphore_signal(
                        ln["credit"], inc=1,
                        device_id=(ln["upstream"],),
                        device_id_type=pl.DeviceIdType.MESH,
                    )
            p = [partial(ln["recv_chunk"](h), ln["lo"]) for ln in lanes]

        fin = (N_HOPS - 1 + 1) % N_SLOTS
        for sub in range(N_SUB):
            rows = pl.ds(sub * m_sub, m_sub)
            rows_v = slice(sub * m_sub, (sub + 1) * m_sub)
            for i, ln in enumerate(lanes):
                descs[i][N_HOPS - 1][sub].wait_recv()
                out_ref[rows, ln["lo"]:ln["lo"] + nh] = (
                    p[i][rows_v, :]
                    + ln["comm"][fin, rows, :].astype(jnp.float32))
        for i, ln in enumerate(lanes):
            for sub in range(N_SUB):
                descs[i][N_HOPS - 1][sub].wait_send()

    return pl.pallas_call(
        body,
        out_shape=jax.ShapeDtypeStruct((m_per, n), jnp.float32),
        in_specs=[
            pl.BlockSpec(memory_space=pltpu.SMEM),
            pl.BlockSpec(memory_space=pltpu.SMEM),
            pl.BlockSpec(memory_space=pltpu.VMEM),
            pl.BlockSpec(memory_space=pltpu.VMEM),
        ],
        out_specs=pl.BlockSpec(memory_space=pltpu.VMEM),
        scratch_shapes=[
            pltpu.VMEM((N_SLOTS, m_per, nh), jnp.bfloat16),
            pltpu.VMEM((N_SLOTS, m_per, nh), jnp.bfloat16),
            pltpu.SemaphoreType.DMA((N_SLOTS, N_SUB)),
            pltpu.SemaphoreType.DMA((N_SLOTS, N_SUB)),
            pltpu.SemaphoreType.DMA((N_SLOTS, N_SUB)),
            pltpu.SemaphoreType.DMA((N_SLOTS, N_SUB)),
            pltpu.SemaphoreType.REGULAR,
            pltpu.SemaphoreType.REGULAR,
        ],
        compiler_params=pltpu.CompilerParams(collective_id=0),
    )(scalars, mesh_of_ring, xb, wb)
